# Initial kernel scaffold; baseline (speedup 1.0000x reference)
import jax
import jax.numpy as jnp
from jax.experimental import pallas as pl


def kernel(x, edge_index, W1, b1, W2, b2, W3, b3):
    raise NotImplementedError("write your pallas kernel here")


# speedup vs baseline: 4.0291x; 4.0291x over previous
import functools

import jax
import jax.numpy as jnp
import numpy as np
from jax import lax
from jax.experimental import pallas as pl
from jax.experimental.pallas import tpu as pltpu
from jax.experimental.pallas import tpu_sc as plsc

N = 10000
NPAD = 10240
E = 320000
D = 128

NC = 2
NS = 16
NW = NC * NS
EPT = E // NW
CHUNK = 80
NCHUNK = EPT // CHUNK
RPT = NPAD // NS
HROWS = NPAD // 16

_mesh = plsc.VectorSubcoreMesh(
    core_axis_name="c", subcore_axis_name="s", num_cores=NC, num_subcores=NS)


@functools.partial(
    pl.kernel,
    out_type=jax.ShapeDtypeStruct((NC, NPAD, D), jnp.float32),
    mesh=_mesh,
    scratch_types=[
        pltpu.VMEM((CHUNK,), jnp.int32),
        pltpu.VMEM((CHUNK,), jnp.int32),
        pltpu.VMEM((CHUNK, D), jnp.float32),
        pltpu.VMEM((CHUNK, D), jnp.float32),
        pltpu.VMEM_SHARED((NPAD, D), jnp.float32),
    ],
)
def _deg(src_hbm, dst_hbm, oness_hbm, onesd_hbm, zeros_hbm, out_hbm,
         idx_s, idx_d, ones_s, ones_d, sh):
    c = lax.axis_index("c")
    s = lax.axis_index("s")
    base = (c * NS + s) * EPT

    pltpu.sync_copy(oness_hbm, ones_s)
    pltpu.sync_copy(onesd_hbm, ones_d)
    pltpu.sync_copy(zeros_hbm, sh.at[pl.ds(s * RPT, RPT)])
    plsc.subcore_barrier()

    def acc(j, carry):
        e0 = base + j * CHUNK
        pltpu.sync_copy(src_hbm.at[pl.ds(e0, CHUNK)], idx_s)
        pltpu.sync_copy(dst_hbm.at[pl.ds(e0, CHUNK)], idx_d)
        pltpu.sync_copy(ones_s, sh.at[idx_s], add=True)
        pltpu.sync_copy(ones_d, sh.at[idx_d], add=True)
        return carry

    lax.fori_loop(0, NCHUNK, acc, 0)
    plsc.subcore_barrier()

    pltpu.sync_copy(sh.at[pl.ds(s * RPT, RPT)],
                    out_hbm.at[c, pl.ds(s * RPT, RPT)])


@functools.partial(
    pl.kernel,
    out_type=jax.ShapeDtypeStruct((NC, NPAD, D), jnp.float32),
    mesh=_mesh,
    scratch_types=[
        pltpu.VMEM((CHUNK,), jnp.int32),
        pltpu.VMEM((CHUNK,), jnp.int32),
        pltpu.VMEM((CHUNK, D), jnp.float32),
        pltpu.VMEM_SHARED((NPAD, D), jnp.float32),
        pltpu.SemaphoreType.DMA,
    ],
)
def _agg(h_hbm, src_hbm, dst_hbm, zeros_hbm, out_hbm,
         idx_g, idx_sc, rows, agg_sh, sem):
    c = lax.axis_index("c")
    s = lax.axis_index("s")

    pltpu.sync_copy(zeros_hbm, agg_sh.at[pl.ds(s * RPT, RPT)])
    plsc.subcore_barrier()

    base = (c * NS + s) * EPT

    def step(j, carry):
        e0 = base + j * CHUNK
        pltpu.sync_copy(src_hbm.at[pl.ds(e0, CHUNK)], idx_g)
        pltpu.sync_copy(dst_hbm.at[pl.ds(e0, CHUNK)], idx_sc)
        pltpu.async_copy(h_hbm.at[idx_g], rows, sem).wait()
        pltpu.sync_copy(rows, agg_sh.at[idx_sc], add=True)
        return carry

    lax.fori_loop(0, NCHUNK, step, 0)
    plsc.subcore_barrier()
    pltpu.sync_copy(agg_sh.at[pl.ds(s * RPT, RPT)],
                    out_hbm.at[c, pl.ds(s * RPT, RPT)])


def _mm1_body(x_ref, on_ref, w_ref, o_ref):
    o_ref[...] = jnp.dot(x_ref[...] * on_ref[...], w_ref[...],
                         preferred_element_type=jnp.float32)


def _layer_body(a_ref, inn_ref, b_ref, on_ref, w_ref, o_ref):
    p = a_ref[0] + a_ref[1]
    h = jnp.maximum(p * inn_ref[...] + b_ref[...], 0.0)
    o_ref[...] = jnp.dot(h * on_ref[...], w_ref[...],
                         preferred_element_type=jnp.float32)


def _final_body(a_ref, inn_ref, b_ref, o_ref):
    i = pl.program_id(0)
    p = (a_ref[0] + a_ref[1]) * inn_ref[...]
    part = jnp.sum(p, axis=0, keepdims=True)

    @pl.when(i == 0)
    def _():
        o_ref[...] = jnp.zeros_like(o_ref)

    o_ref[...] += part

    @pl.when(i == pl.num_programs(0) - 1)
    def _():
        o_ref[...] = o_ref[...] * (1.0 / N) + b_ref[...]


_R = 1024
_G = NPAD // _R


def _mm1(x, on, w):
    return pl.pallas_call(
        _mm1_body,
        grid=(_G,),
        in_specs=[
            pl.BlockSpec((_R, D), lambda i: (i, 0)),
            pl.BlockSpec((_R, 1), lambda i: (i, 0)),
            pl.BlockSpec((D, D), lambda i: (0, 0)),
        ],
        out_specs=pl.BlockSpec((_R, D), lambda i: (i, 0)),
        out_shape=jax.ShapeDtypeStruct((NPAD, D), jnp.float32),
    )(x, on, w)


def _layer(a, inn, b, on, w):
    return pl.pallas_call(
        _layer_body,
        grid=(_G,),
        in_specs=[
            pl.BlockSpec((NC, _R, D), lambda i: (0, i, 0)),
            pl.BlockSpec((_R, 1), lambda i: (i, 0)),
            pl.BlockSpec((1, D), lambda i: (0, 0)),
            pl.BlockSpec((_R, 1), lambda i: (i, 0)),
            pl.BlockSpec((D, D), lambda i: (0, 0)),
        ],
        out_specs=pl.BlockSpec((_R, D), lambda i: (i, 0)),
        out_shape=jax.ShapeDtypeStruct((NPAD, D), jnp.float32),
    )(a, inn, b, on, w)


def _final(a, inn, b):
    return pl.pallas_call(
        _final_body,
        grid=(_G,),
        in_specs=[
            pl.BlockSpec((NC, _R, D), lambda i: (0, i, 0)),
            pl.BlockSpec((_R, 1), lambda i: (i, 0)),
            pl.BlockSpec((1, D), lambda i: (0, 0)),
        ],
        out_specs=pl.BlockSpec((1, D), lambda i: (0, 0)),
        out_shape=jax.ShapeDtypeStruct((1, D), jnp.float32),
    )(a, inn, b)


def kernel(x, edge_index, W1, b1, W2, b2, W3, b3):
    src = edge_index[0]
    dst = edge_index[1]
    col = jnp.arange(D) < (D // 2)
    ones_s = jnp.where(col, 1.0, 0.0)[None, :] * jnp.ones((CHUNK, 1))
    ones_d = jnp.where(col, 0.0, 1.0)[None, :] * jnp.ones((CHUNK, 1))
    zeros = jnp.zeros((RPT, D), jnp.float32)

    deg = _deg(src, dst, ones_s.astype(jnp.float32),
               ones_d.astype(jnp.float32), zeros)
    out_deg = (deg[0, :, 0] + deg[1, :, 0]).reshape(NPAD, 1)
    in_deg = (deg[0, :, D // 2] + deg[1, :, D // 2]).reshape(NPAD, 1)
    out_norm = jax.lax.rsqrt(jnp.clip(out_deg, 1.0, None))
    in_norm = jax.lax.rsqrt(jnp.clip(in_deg, 1.0, None))

    x_pad = jnp.concatenate(
        [x, jnp.zeros((NPAD - N, D), jnp.float32)], axis=0)

    h1 = _mm1(x_pad, out_norm, W1)
    a1 = _agg(h1, src, dst, zeros)
    h2 = _layer(a1, in_norm, b1.reshape(1, D), out_norm, W2)
    a2 = _agg(h2, src, dst, zeros)
    h3 = _layer(a2, in_norm, b2.reshape(1, D), out_norm, W3)
    a3 = _agg(h3, src, dst, zeros)
    out = _final(a3, in_norm, b3.reshape(1, D))
    return out.reshape(D)


# speedup vs baseline: 9.0116x; 2.2367x over previous
import functools

import jax
import jax.numpy as jnp
from jax import lax
from jax.experimental import pallas as pl
from jax.experimental.pallas import tpu as pltpu
from jax.experimental.pallas import tpu_sc as plsc

N = 10000
NPAD = 10240
E = 320000
EPAD = 327680
D = 128

NC = 2
NS = 16
NW = NC * NS
EPT = EPAD // NW
CHUNK = 128
NCH = EPT // CHUNK
RPT = NPAD // NS
HH = NCH // 2

_mesh = plsc.VectorSubcoreMesh(
    core_axis_name="c", subcore_axis_name="s", num_cores=NC, num_subcores=NS)


@functools.partial(
    pl.kernel,
    out_type=jax.ShapeDtypeStruct((NC, NPAD, D), jnp.float32),
    mesh=_mesh,
    scratch_types=[
        pltpu.VMEM((HH, CHUNK), jnp.int32),
        pltpu.VMEM((HH, CHUNK), jnp.int32),
        pltpu.VMEM((CHUNK, D), jnp.float32),
        pltpu.VMEM((CHUNK, D), jnp.float32),
        pltpu.VMEM_SHARED((NPAD, D), jnp.float32),
        pltpu.SemaphoreType.DMA,
    ],
)
def _deg(src2d, dst2d, oness_hbm, onesd_hbm, zeros_hbm, out_hbm,
         idxs, idxd, ones_s, ones_d, sh, sem):
    c = lax.axis_index("c")
    s = lax.axis_index("s")
    r0 = (c * NS + s) * NCH

    pltpu.sync_copy(oness_hbm, ones_s)
    pltpu.sync_copy(onesd_hbm, ones_d)
    pltpu.sync_copy(zeros_hbm, sh.at[pl.ds(s * RPT, RPT)])
    plsc.subcore_barrier()

    def fire(j, carry):
        pltpu.async_copy(ones_s, sh.at[idxs.at[j]], sem, add=True)
        pltpu.async_copy(ones_d, sh.at[idxd.at[j]], sem, add=True)
        return carry

    def drain(j, carry):
        pltpu.make_async_copy(ones_s, sh.at[idxs.at[0]], sem).wait()
        pltpu.make_async_copy(ones_d, sh.at[idxd.at[0]], sem).wait()
        return carry

    for half in range(2):
        pltpu.sync_copy(src2d.at[pl.ds(r0 + half * HH, HH)], idxs)
        pltpu.sync_copy(dst2d.at[pl.ds(r0 + half * HH, HH)], idxd)
        lax.fori_loop(0, HH, fire, 0)
        lax.fori_loop(0, HH, drain, 0)

    plsc.subcore_barrier()

    pltpu.sync_copy(sh.at[pl.ds(s * RPT, RPT)],
                    out_hbm.at[c, pl.ds(s * RPT, RPT)])


@functools.partial(
    pl.kernel,
    out_type=jax.ShapeDtypeStruct((NC, NPAD, D), jnp.float32),
    mesh=_mesh,
    scratch_types=[
        pltpu.VMEM((HH, CHUNK), jnp.int32),
        pltpu.VMEM((HH, CHUNK), jnp.int32),
        pltpu.VMEM((CHUNK, D), jnp.float32),
        pltpu.VMEM((CHUNK, D), jnp.float32),
        pltpu.VMEM_SHARED((NPAD, D), jnp.float32),
        pltpu.SemaphoreType.DMA,
        pltpu.SemaphoreType.DMA,
    ],
)
def _agg(h_hbm, src2d, dst2d, zeros_hbm, out_hbm,
         idxs, idxd, rows0, rows1, sh, sem0, sem1):
    c = lax.axis_index("c")
    s = lax.axis_index("s")
    r0 = (c * NS + s) * NCH

    pltpu.sync_copy(zeros_hbm, sh.at[pl.ds(s * RPT, RPT)])
    plsc.subcore_barrier()

    def step(t, carry):
        j0 = 2 * t
        pltpu.make_async_copy(h_hbm.at[idxs.at[0]], rows0, sem0).wait()
        pltpu.sync_copy(rows0, sh.at[idxd.at[j0]], add=True)

        @pl.when(j0 + 2 < HH)
        def _():
            pltpu.async_copy(h_hbm.at[idxs.at[j0 + 2]], rows0, sem0)

        pltpu.make_async_copy(h_hbm.at[idxs.at[1]], rows1, sem1).wait()
        pltpu.sync_copy(rows1, sh.at[idxd.at[j0 + 1]], add=True)

        @pl.when(j0 + 3 < HH)
        def _():
            pltpu.async_copy(h_hbm.at[idxs.at[j0 + 3]], rows1, sem1)

        return carry

    for half in range(2):
        pltpu.sync_copy(src2d.at[pl.ds(r0 + half * HH, HH)], idxs)
        pltpu.sync_copy(dst2d.at[pl.ds(r0 + half * HH, HH)], idxd)
        pltpu.async_copy(h_hbm.at[idxs.at[0]], rows0, sem0)
        pltpu.async_copy(h_hbm.at[idxs.at[1]], rows1, sem1)
        lax.fori_loop(0, HH // 2, step, 0)

    plsc.subcore_barrier()
    pltpu.sync_copy(sh.at[pl.ds(s * RPT, RPT)],
                    out_hbm.at[c, pl.ds(s * RPT, RPT)])


def _mm1_body(x_ref, on_ref, w_ref, o_ref):
    o_ref[...] = jnp.dot(x_ref[...] * on_ref[...], w_ref[...],
                         preferred_element_type=jnp.float32)


def _layer_body(a_ref, inn_ref, b_ref, on_ref, w_ref, o_ref):
    p = a_ref[0] + a_ref[1]
    h = jnp.maximum(p * inn_ref[...] + b_ref[...], 0.0)
    o_ref[...] = jnp.dot(h * on_ref[...], w_ref[...],
                         preferred_element_type=jnp.float32)


def _final_body(a_ref, inn_ref, b_ref, o_ref):
    i = pl.program_id(0)
    rows = lax.broadcasted_iota(jnp.int32, (_R, 1), 0) + i * _R
    inn = jnp.where(rows < N, inn_ref[...], 0.0)
    p = (a_ref[0] + a_ref[1]) * inn
    part = jnp.sum(p, axis=0, keepdims=True)

    @pl.when(i == 0)
    def _():
        o_ref[...] = jnp.zeros_like(o_ref)

    o_ref[...] += part

    @pl.when(i == pl.num_programs(0) - 1)
    def _():
        o_ref[...] = o_ref[...] * (1.0 / N) + b_ref[...]


_R = 1024
_G = NPAD // _R


def _mm1(x, on, w):
    return pl.pallas_call(
        _mm1_body,
        grid=(_G,),
        in_specs=[
            pl.BlockSpec((_R, D), lambda i: (i, 0)),
            pl.BlockSpec((_R, 1), lambda i: (i, 0)),
            pl.BlockSpec((D, D), lambda i: (0, 0)),
        ],
        out_specs=pl.BlockSpec((_R, D), lambda i: (i, 0)),
        out_shape=jax.ShapeDtypeStruct((NPAD, D), jnp.float32),
    )(x, on, w)


def _layer(a, inn, b, on, w):
    return pl.pallas_call(
        _layer_body,
        grid=(_G,),
        in_specs=[
            pl.BlockSpec((NC, _R, D), lambda i: (0, i, 0)),
            pl.BlockSpec((_R, 1), lambda i: (i, 0)),
            pl.BlockSpec((1, D), lambda i: (0, 0)),
            pl.BlockSpec((_R, 1), lambda i: (i, 0)),
            pl.BlockSpec((D, D), lambda i: (0, 0)),
        ],
        out_specs=pl.BlockSpec((_R, D), lambda i: (i, 0)),
        out_shape=jax.ShapeDtypeStruct((NPAD, D), jnp.float32),
    )(a, inn, b, on, w)


def _final(a, inn, b):
    return pl.pallas_call(
        _final_body,
        grid=(_G,),
        in_specs=[
            pl.BlockSpec((NC, _R, D), lambda i: (0, i, 0)),
            pl.BlockSpec((_R, 1), lambda i: (i, 0)),
            pl.BlockSpec((1, D), lambda i: (0, 0)),
        ],
        out_specs=pl.BlockSpec((1, D), lambda i: (0, 0)),
        out_shape=jax.ShapeDtypeStruct((1, D), jnp.float32),
    )(a, inn, b)


def kernel(x, edge_index, W1, b1, W2, b2, W3, b3):
    pad_ids = N + (jnp.arange(EPAD - E, dtype=jnp.int32) % (NPAD - N))
    src = jnp.concatenate([edge_index[0], pad_ids]).reshape(EPAD // CHUNK,
                                                            CHUNK)
    dst = jnp.concatenate([edge_index[1], pad_ids]).reshape(EPAD // CHUNK,
                                                            CHUNK)

    col = jnp.arange(D) < (D // 2)
    ones_s = jnp.broadcast_to(jnp.where(col, 1.0, 0.0), (CHUNK, D))
    ones_d = jnp.broadcast_to(jnp.where(col, 0.0, 1.0), (CHUNK, D))
    zeros = jnp.zeros((RPT, D), jnp.float32)

    deg = _deg(src, dst, ones_s.astype(jnp.float32),
               ones_d.astype(jnp.float32), zeros)
    out_deg = (deg[0, :, 0] + deg[1, :, 0]).reshape(NPAD, 1)
    in_deg = (deg[0, :, D // 2] + deg[1, :, D // 2]).reshape(NPAD, 1)
    out_norm = jax.lax.rsqrt(jnp.clip(out_deg, 1.0, None))
    in_norm = jax.lax.rsqrt(jnp.clip(in_deg, 1.0, None))

    x_pad = jnp.concatenate(
        [x, jnp.zeros((NPAD - N, D), jnp.float32)], axis=0)

    h1 = _mm1(x_pad, out_norm, W1)
    a1 = _agg(h1, src, dst, zeros)
    h2 = _layer(a1, in_norm, b1.reshape(1, D), out_norm, W2)
    a2 = _agg(h2, src, dst, zeros)
    h3 = _layer(a2, in_norm, b2.reshape(1, D), out_norm, W3)
    a3 = _agg(h3, src, dst, zeros)
    out = _final(a3, in_norm, b3.reshape(1, D))
    return out.reshape(D)


# speedup vs baseline: 11.6593x; 1.2938x over previous
import functools

import jax
import jax.numpy as jnp
from jax import lax
from jax.experimental import pallas as pl
from jax.experimental.pallas import tpu as pltpu
from jax.experimental.pallas import tpu_sc as plsc

N = 10000
NPAD = 10240
E = 320000
EPAD = 327680
D = 128

NC = 2
NS = 16
NW = NC * NS
EPT = EPAD // NW
CHUNK = 128
NCH = EPT // CHUNK
RPT = NPAD // NS
HH = NCH // 2

_mesh = plsc.VectorSubcoreMesh(
    core_axis_name="c", subcore_axis_name="s", num_cores=NC, num_subcores=NS)


@functools.partial(
    pl.kernel,
    out_type=[
        jax.ShapeDtypeStruct((NC, NPAD), jnp.float32),
        jax.ShapeDtypeStruct((NC, NPAD), jnp.float32),
    ],
    mesh=_mesh,
    scratch_types=[
        pltpu.VMEM((NCH, CHUNK), jnp.int32),
        pltpu.VMEM((NCH, CHUNK), jnp.int32),
        pltpu.VMEM((CHUNK,), jnp.float32),
        pltpu.VMEM((RPT,), jnp.float32),
        pltpu.VMEM_SHARED((NPAD,), jnp.float32),
        pltpu.VMEM_SHARED((NPAD,), jnp.float32),
        pltpu.SemaphoreType.DMA,
    ],
)
def _deg(src2d, dst2d, outs_hbm, outd_hbm,
         idxs, idxd, ones_v, zbuf, sh_s, sh_d, sem):
    c = lax.axis_index("c")
    s = lax.axis_index("s")
    r0 = (c * NS + s) * NCH

    one = jnp.ones((16,), jnp.float32)
    zero = jnp.zeros((16,), jnp.float32)
    for k in range(CHUNK // 16):
        ones_v[pl.ds(k * 16, 16)] = one

    def zrow(r, carry):
        zbuf[pl.ds(r * 16, 16)] = zero
        return carry

    lax.fori_loop(0, RPT // 16, zrow, 0)
    pltpu.sync_copy(src2d.at[pl.ds(r0, NCH)], idxs)
    pltpu.sync_copy(dst2d.at[pl.ds(r0, NCH)], idxd)
    pltpu.sync_copy(zbuf, sh_s.at[pl.ds(s * RPT, RPT)])
    pltpu.sync_copy(zbuf, sh_d.at[pl.ds(s * RPT, RPT)])
    plsc.subcore_barrier()

    def fire(j, carry):
        pltpu.async_copy(ones_v, sh_s.at[idxs.at[j]], sem, add=True)
        pltpu.async_copy(ones_v, sh_d.at[idxd.at[j]], sem, add=True)
        return carry

    def drain(j, carry):
        pltpu.make_async_copy(ones_v, sh_s.at[idxs.at[0]], sem).wait()
        pltpu.make_async_copy(ones_v, sh_d.at[idxd.at[0]], sem).wait()
        return carry

    lax.fori_loop(0, NCH, fire, 0)
    lax.fori_loop(0, NCH, drain, 0)
    plsc.subcore_barrier()

    pltpu.sync_copy(sh_s.at[pl.ds(s * RPT, RPT)],
                    outs_hbm.at[c, pl.ds(s * RPT, RPT)])
    pltpu.sync_copy(sh_d.at[pl.ds(s * RPT, RPT)],
                    outd_hbm.at[c, pl.ds(s * RPT, RPT)])


@functools.partial(
    pl.kernel,
    out_type=jax.ShapeDtypeStruct((NC, NPAD, D), jnp.float32),
    mesh=_mesh,
    scratch_types=[
        pltpu.VMEM((HH, CHUNK), jnp.int32),
        pltpu.VMEM((HH, CHUNK), jnp.int32),
        pltpu.VMEM((CHUNK, D), jnp.float32),
        pltpu.VMEM((CHUNK, D), jnp.float32),
        pltpu.VMEM_SHARED((NPAD, D), jnp.float32),
        pltpu.SemaphoreType.DMA,
        pltpu.SemaphoreType.DMA,
    ],
)
def _agg(h_hbm, src2d, dst2d, zeros_hbm, out_hbm,
         idxs, idxd, rows0, rows1, sh, sem0, sem1):
    c = lax.axis_index("c")
    s = lax.axis_index("s")
    r0 = (c * NS + s) * NCH

    pltpu.sync_copy(zeros_hbm, sh.at[pl.ds(s * RPT, RPT)])
    plsc.subcore_barrier()

    def step(t, carry):
        j0 = 2 * t
        pltpu.make_async_copy(h_hbm.at[idxs.at[0]], rows0, sem0).wait()
        pltpu.sync_copy(rows0, sh.at[idxd.at[j0]], add=True)

        @pl.when(j0 + 2 < HH)
        def _():
            pltpu.async_copy(h_hbm.at[idxs.at[j0 + 2]], rows0, sem0)

        pltpu.make_async_copy(h_hbm.at[idxs.at[1]], rows1, sem1).wait()
        pltpu.sync_copy(rows1, sh.at[idxd.at[j0 + 1]], add=True)

        @pl.when(j0 + 3 < HH)
        def _():
            pltpu.async_copy(h_hbm.at[idxs.at[j0 + 3]], rows1, sem1)

        return carry

    for half in range(2):
        pltpu.sync_copy(src2d.at[pl.ds(r0 + half * HH, HH)], idxs)
        pltpu.sync_copy(dst2d.at[pl.ds(r0 + half * HH, HH)], idxd)
        pltpu.async_copy(h_hbm.at[idxs.at[0]], rows0, sem0)
        pltpu.async_copy(h_hbm.at[idxs.at[1]], rows1, sem1)
        lax.fori_loop(0, HH // 2, step, 0)

    plsc.subcore_barrier()
    pltpu.sync_copy(sh.at[pl.ds(s * RPT, RPT)],
                    out_hbm.at[c, pl.ds(s * RPT, RPT)])


def _mm1_body(x_ref, on_ref, w_ref, o_ref):
    o_ref[...] = jnp.dot(x_ref[...] * on_ref[...], w_ref[...],
                         preferred_element_type=jnp.float32)


def _layer_body(a_ref, inn_ref, b_ref, on_ref, w_ref, o_ref):
    p = a_ref[0] + a_ref[1]
    h = jnp.maximum(p * inn_ref[...] + b_ref[...], 0.0)
    o_ref[...] = jnp.dot(h * on_ref[...], w_ref[...],
                         preferred_element_type=jnp.float32)


def _final_body(a_ref, inn_ref, b_ref, o_ref):
    i = pl.program_id(0)
    rows = lax.broadcasted_iota(jnp.int32, (_R, 1), 0) + i * _R
    inn = jnp.where(rows < N, inn_ref[...], 0.0)
    p = (a_ref[0] + a_ref[1]) * inn
    part = jnp.sum(p, axis=0, keepdims=True)

    @pl.when(i == 0)
    def _():
        o_ref[...] = jnp.zeros_like(o_ref)

    o_ref[...] += part

    @pl.when(i == pl.num_programs(0) - 1)
    def _():
        o_ref[...] = o_ref[...] * (1.0 / N) + b_ref[...]


_R = 1024
_G = NPAD // _R


def _mm1(x, on, w):
    return pl.pallas_call(
        _mm1_body,
        grid=(_G,),
        in_specs=[
            pl.BlockSpec((_R, D), lambda i: (i, 0)),
            pl.BlockSpec((_R, 1), lambda i: (i, 0)),
            pl.BlockSpec((D, D), lambda i: (0, 0)),
        ],
        out_specs=pl.BlockSpec((_R, D), lambda i: (i, 0)),
        out_shape=jax.ShapeDtypeStruct((NPAD, D), jnp.float32),
    )(x, on, w)


def _layer(a, inn, b, on, w):
    return pl.pallas_call(
        _layer_body,
        grid=(_G,),
        in_specs=[
            pl.BlockSpec((NC, _R, D), lambda i: (0, i, 0)),
            pl.BlockSpec((_R, 1), lambda i: (i, 0)),
            pl.BlockSpec((1, D), lambda i: (0, 0)),
            pl.BlockSpec((_R, 1), lambda i: (i, 0)),
            pl.BlockSpec((D, D), lambda i: (0, 0)),
        ],
        out_specs=pl.BlockSpec((_R, D), lambda i: (i, 0)),
        out_shape=jax.ShapeDtypeStruct((NPAD, D), jnp.float32),
    )(a, inn, b, on, w)


def _final(a, inn, b):
    return pl.pallas_call(
        _final_body,
        grid=(_G,),
        in_specs=[
            pl.BlockSpec((NC, _R, D), lambda i: (0, i, 0)),
            pl.BlockSpec((_R, 1), lambda i: (i, 0)),
            pl.BlockSpec((1, D), lambda i: (0, 0)),
        ],
        out_specs=pl.BlockSpec((1, D), lambda i: (0, 0)),
        out_shape=jax.ShapeDtypeStruct((1, D), jnp.float32),
    )(a, inn, b)


def kernel(x, edge_index, W1, b1, W2, b2, W3, b3):
    pad_ids = N + (jnp.arange(EPAD - E, dtype=jnp.int32) % (NPAD - N))
    src = jnp.concatenate([edge_index[0], pad_ids]).reshape(EPAD // CHUNK,
                                                            CHUNK)
    dst = jnp.concatenate([edge_index[1], pad_ids]).reshape(EPAD // CHUNK,
                                                            CHUNK)

    zeros = jnp.zeros((RPT, D), jnp.float32)

    degs, degd = _deg(src, dst)
    out_deg = (degs[0] + degs[1]).reshape(NPAD, 1)
    in_deg = (degd[0] + degd[1]).reshape(NPAD, 1)
    out_norm = jax.lax.rsqrt(jnp.clip(out_deg, 1.0, None))
    in_norm = jax.lax.rsqrt(jnp.clip(in_deg, 1.0, None))

    x_pad = jnp.concatenate(
        [x, jnp.zeros((NPAD - N, D), jnp.float32)], axis=0)

    h1 = _mm1(x_pad, out_norm, W1)
    a1 = _agg(h1, src, dst, zeros)
    h2 = _layer(a1, in_norm, b1.reshape(1, D), out_norm, W2)
    a2 = _agg(h2, src, dst, zeros)
    h3 = _layer(a2, in_norm, b2.reshape(1, D), out_norm, W3)
    a3 = _agg(h3, src, dst, zeros)
    out = _final(a3, in_norm, b3.reshape(1, D))
    return out.reshape(D)


# speedup vs baseline: 12.2990x; 1.0549x over previous
import functools

import jax
import jax.numpy as jnp
from jax import lax
from jax.experimental import pallas as pl
from jax.experimental.pallas import tpu as pltpu
from jax.experimental.pallas import tpu_sc as plsc

N = 10000
NPAD = 10240
E = 320000
EPAD = 327680
D = 128

NC = 2
NS = 16
NW = NC * NS
EPT = EPAD // NW
CHUNK = 128
NCH = EPT // CHUNK
RPT = NPAD // NS
NB = 4
CH2 = 64
NCH2 = EPT // CH2
HH2 = NCH2 // 4

_mesh = plsc.VectorSubcoreMesh(
    core_axis_name="c", subcore_axis_name="s", num_cores=NC, num_subcores=NS)


@functools.partial(
    pl.kernel,
    out_type=[
        jax.ShapeDtypeStruct((NC, NPAD), jnp.float32),
        jax.ShapeDtypeStruct((NC, NPAD), jnp.float32),
    ],
    mesh=_mesh,
    scratch_types=[
        pltpu.VMEM((NCH, CHUNK), jnp.int32),
        pltpu.VMEM((NCH, CHUNK), jnp.int32),
        pltpu.VMEM((CHUNK,), jnp.float32),
        pltpu.VMEM((RPT,), jnp.float32),
        pltpu.VMEM_SHARED((NPAD,), jnp.float32),
        pltpu.VMEM_SHARED((NPAD,), jnp.float32),
        pltpu.SemaphoreType.DMA,
    ],
)
def _deg(src2d, dst2d, outs_hbm, outd_hbm,
         idxs, idxd, ones_v, zbuf, sh_s, sh_d, sem):
    c = lax.axis_index("c")
    s = lax.axis_index("s")
    r0 = (c * NS + s) * NCH

    one = jnp.ones((16,), jnp.float32)
    zero = jnp.zeros((16,), jnp.float32)
    for k in range(CHUNK // 16):
        ones_v[pl.ds(k * 16, 16)] = one

    def zrow(r, carry):
        zbuf[pl.ds(r * 16, 16)] = zero
        return carry

    lax.fori_loop(0, RPT // 16, zrow, 0)
    pltpu.sync_copy(src2d.at[pl.ds(r0, NCH)], idxs)
    pltpu.sync_copy(dst2d.at[pl.ds(r0, NCH)], idxd)
    pltpu.sync_copy(zbuf, sh_s.at[pl.ds(s * RPT, RPT)])
    pltpu.sync_copy(zbuf, sh_d.at[pl.ds(s * RPT, RPT)])
    plsc.subcore_barrier()

    def fire(j, carry):
        pltpu.async_copy(ones_v, sh_s.at[idxs.at[j]], sem, add=True)
        pltpu.async_copy(ones_v, sh_d.at[idxd.at[j]], sem, add=True)
        return carry

    def drain(j, carry):
        pltpu.make_async_copy(ones_v, sh_s.at[idxs.at[0]], sem).wait()
        pltpu.make_async_copy(ones_v, sh_d.at[idxd.at[0]], sem).wait()
        return carry

    lax.fori_loop(0, NCH, fire, 0)
    lax.fori_loop(0, NCH, drain, 0)
    plsc.subcore_barrier()

    pltpu.sync_copy(sh_s.at[pl.ds(s * RPT, RPT)],
                    outs_hbm.at[c, pl.ds(s * RPT, RPT)])
    pltpu.sync_copy(sh_d.at[pl.ds(s * RPT, RPT)],
                    outd_hbm.at[c, pl.ds(s * RPT, RPT)])


@functools.partial(
    pl.kernel,
    out_type=jax.ShapeDtypeStruct((NC, NPAD, D), jnp.float32),
    mesh=_mesh,
    scratch_types=[
        pltpu.VMEM((HH2, CH2), jnp.int32),
        pltpu.VMEM((HH2, CH2), jnp.int32),
        pltpu.VMEM((CH2, D), jnp.float32),
        pltpu.VMEM((CH2, D), jnp.float32),
        pltpu.VMEM((CH2, D), jnp.float32),
        pltpu.VMEM((CH2, D), jnp.float32),
        pltpu.VMEM_SHARED((NPAD, D), jnp.float32),
        pltpu.SemaphoreType.DMA,
        pltpu.SemaphoreType.DMA,
        pltpu.SemaphoreType.DMA,
        pltpu.SemaphoreType.DMA,
    ],
)
def _agg(h_hbm, src2d, dst2d, zeros_hbm, out_hbm,
         idxs, idxd, rows0, rows1, rows2, rows3,
         sh, sem0, sem1, sem2, sem3):
    c = lax.axis_index("c")
    s = lax.axis_index("s")
    r0 = (c * NS + s) * NCH2
    rows = [rows0, rows1, rows2, rows3]
    sems = [sem0, sem1, sem2, sem3]

    pltpu.sync_copy(zeros_hbm, sh.at[pl.ds(s * RPT, RPT)])
    plsc.subcore_barrier()

    def step(t, carry):
        for ss in range(NB):
            j = NB * t + ss
            pltpu.make_async_copy(
                h_hbm.at[idxs.at[0]], rows[ss], sems[ss]).wait()
            pltpu.sync_copy(rows[ss], sh.at[idxd.at[j]], add=True)

            @pl.when(j + NB < HH2)
            def _():
                pltpu.async_copy(h_hbm.at[idxs.at[j + NB]], rows[ss],
                                 sems[ss])

        return carry

    for half in range(4):
        pltpu.sync_copy(src2d.at[pl.ds(r0 + half * HH2, HH2)], idxs)
        pltpu.sync_copy(dst2d.at[pl.ds(r0 + half * HH2, HH2)], idxd)
        for b in range(NB):
            pltpu.async_copy(h_hbm.at[idxs.at[b]], rows[b], sems[b])
        lax.fori_loop(0, HH2 // NB, step, 0)

    plsc.subcore_barrier()
    pltpu.sync_copy(sh.at[pl.ds(s * RPT, RPT)],
                    out_hbm.at[c, pl.ds(s * RPT, RPT)])


def _mm1_body(x_ref, on_ref, w_ref, o_ref):
    o_ref[...] = jnp.dot(x_ref[...] * on_ref[...], w_ref[...],
                         preferred_element_type=jnp.float32)


def _layer_body(a_ref, inn_ref, b_ref, on_ref, w_ref, o_ref):
    p = a_ref[0] + a_ref[1]
    h = jnp.maximum(p * inn_ref[...] + b_ref[...], 0.0)
    o_ref[...] = jnp.dot(h * on_ref[...], w_ref[...],
                         preferred_element_type=jnp.float32)


def _final_body(a_ref, inn_ref, b_ref, o_ref):
    i = pl.program_id(0)
    rows = lax.broadcasted_iota(jnp.int32, (_R, 1), 0) + i * _R
    inn = jnp.where(rows < N, inn_ref[...], 0.0)
    p = (a_ref[0] + a_ref[1]) * inn
    part = jnp.sum(p, axis=0, keepdims=True)

    @pl.when(i == 0)
    def _():
        o_ref[...] = jnp.zeros_like(o_ref)

    o_ref[...] += part

    @pl.when(i == pl.num_programs(0) - 1)
    def _():
        o_ref[...] = o_ref[...] * (1.0 / N) + b_ref[...]


_R = 1024
_G = NPAD // _R


def _mm1(x, on, w):
    return pl.pallas_call(
        _mm1_body,
        grid=(_G,),
        in_specs=[
            pl.BlockSpec((_R, D), lambda i: (i, 0)),
            pl.BlockSpec((_R, 1), lambda i: (i, 0)),
            pl.BlockSpec((D, D), lambda i: (0, 0)),
        ],
        out_specs=pl.BlockSpec((_R, D), lambda i: (i, 0)),
        out_shape=jax.ShapeDtypeStruct((NPAD, D), jnp.float32),
    )(x, on, w)


def _layer(a, inn, b, on, w):
    return pl.pallas_call(
        _layer_body,
        grid=(_G,),
        in_specs=[
            pl.BlockSpec((NC, _R, D), lambda i: (0, i, 0)),
            pl.BlockSpec((_R, 1), lambda i: (i, 0)),
            pl.BlockSpec((1, D), lambda i: (0, 0)),
            pl.BlockSpec((_R, 1), lambda i: (i, 0)),
            pl.BlockSpec((D, D), lambda i: (0, 0)),
        ],
        out_specs=pl.BlockSpec((_R, D), lambda i: (i, 0)),
        out_shape=jax.ShapeDtypeStruct((NPAD, D), jnp.float32),
    )(a, inn, b, on, w)


def _final(a, inn, b):
    return pl.pallas_call(
        _final_body,
        grid=(_G,),
        in_specs=[
            pl.BlockSpec((NC, _R, D), lambda i: (0, i, 0)),
            pl.BlockSpec((_R, 1), lambda i: (i, 0)),
            pl.BlockSpec((1, D), lambda i: (0, 0)),
        ],
        out_specs=pl.BlockSpec((1, D), lambda i: (0, 0)),
        out_shape=jax.ShapeDtypeStruct((1, D), jnp.float32),
    )(a, inn, b)


def kernel(x, edge_index, W1, b1, W2, b2, W3, b3):
    pad_ids = N + (jnp.arange(EPAD - E, dtype=jnp.int32) % (NPAD - N))
    src_f = jnp.concatenate([edge_index[0], pad_ids])
    dst_f = jnp.concatenate([edge_index[1], pad_ids])
    src = src_f.reshape(EPAD // CHUNK, CHUNK)
    dst = dst_f.reshape(EPAD // CHUNK, CHUNK)
    src64 = src_f.reshape(EPAD // CH2, CH2)
    dst64 = dst_f.reshape(EPAD // CH2, CH2)

    zeros = jnp.zeros((RPT, D), jnp.float32)

    degs, degd = _deg(src, dst)
    out_deg = (degs[0] + degs[1]).reshape(NPAD, 1)
    in_deg = (degd[0] + degd[1]).reshape(NPAD, 1)
    out_norm = jax.lax.rsqrt(jnp.clip(out_deg, 1.0, None))
    in_norm = jax.lax.rsqrt(jnp.clip(in_deg, 1.0, None))

    x_pad = jnp.concatenate(
        [x, jnp.zeros((NPAD - N, D), jnp.float32)], axis=0)

    h1 = _mm1(x_pad, out_norm, W1)
    a1 = _agg(h1, src64, dst64, zeros)
    h2 = _layer(a1, in_norm, b1.reshape(1, D), out_norm, W2)
    a2 = _agg(h2, src64, dst64, zeros)
    h3 = _layer(a2, in_norm, b2.reshape(1, D), out_norm, W3)
    a3 = _agg(h3, src64, dst64, zeros)
    out = _final(a3, in_norm, b3.reshape(1, D))
    return out.reshape(D)
